# Initial kernel scaffold; baseline (speedup 1.0000x reference)
import jax
import jax.numpy as jnp
from jax.experimental import pallas as pl


def kernel(guide, source, mask_lr, y_bicubic, var_w, var_b, fe_w1, fe_b1, fe_w2, fe_b2, fe_w3, fe_b3, log_lambda, log_mu):
    raise NotImplementedError("write your pallas kernel here")


# speedup vs baseline: 3.7513x; 3.7513x over previous
from functools import partial

import jax
import jax.numpy as jnp
from jax.experimental import pallas as pl
from jax.experimental.pallas import tpu as pltpu

S = 8
H = 256
W = 256
HW = H * W
B = 4
HL = H // S
WL = W // S

HALO = 4
STRIP = 64
NSTRIP = H // STRIP
WIN = STRIP + 2 * HALO
WINF = WIN * W
STRIPF = STRIP * W

_DOT = partial(jnp.dot, preferred_element_type=jnp.float32,
               precision=jax.lax.Precision.HIGHEST)


def _colshift(z, d, is_w0, is_wlast):
    C = z.shape[0]
    if d == 1:
        s = jnp.concatenate([z[:, 1:], jnp.zeros((C, 1), z.dtype)], axis=1)
        return jnp.where(is_wlast, jnp.float32(0.0), s)
    else:
        s = jnp.concatenate([jnp.zeros((C, 1), z.dtype), z[:, :-1]], axis=1)
        return jnp.where(is_w0, jnp.float32(0.0), s)


def _rowshift(z, d):
    C = z.shape[0]
    if d == 1:
        return jnp.concatenate([z[:, W:], jnp.zeros((C, W), z.dtype)], axis=1)
    else:
        return jnp.concatenate([jnp.zeros((C, W), z.dtype), z[:, :-W]], axis=1)


def _conv_aff_body(mu_ref, x_ref, w1_ref, b1_ref, w2_ref, b2_ref,
                   w3_ref, b3_ref, var_ref, aff_ref):
    neg_inv_mu = -1.0 / mu_ref[0, 0]
    s = pl.program_id(1)
    x = x_ref[0, :, pl.ds(s * STRIPF, WINF)]

    pos = jax.lax.broadcasted_iota(jnp.int32, (1, WINF), 1)
    wpos = pos % W
    is_w0 = wpos == 0
    is_wlast = wpos == (W - 1)
    grow = pos // W + s * STRIP - HALO
    in_image = jnp.logical_and(grow >= 0, grow < H)

    def conv(xin, w_ref, b_ref):
        xc = jnp.concatenate([_colshift(xin, -1, is_w0, is_wlast), xin,
                              _colshift(xin, 1, is_w0, is_wlast)], axis=0)
        z0 = _DOT(w_ref[0], xc)
        z1 = _DOT(w_ref[1], xc)
        z2 = _DOT(w_ref[2], xc)
        return _rowshift(z0, -1) + z1 + _rowshift(z2, 1) + b_ref[...]

    y1 = conv(x, w1_ref, b1_ref)
    var_ref[0] = y1[0:1, HALO * W:HALO * W + STRIPF]
    x1 = jnp.where(in_image, jax.nn.relu(y1[1:33]), jnp.float32(0.0))
    x2 = jnp.where(in_image, jax.nn.relu(conv(x1, w2_ref, b2_ref)),
                   jnp.float32(0.0))
    f = conv(x2, w3_ref, b3_ref)

    sl = lambda z: z[:, HALO * W:HALO * W + STRIPF]
    fc = sl(f)
    nu = sl(_rowshift(f, -1))
    nd = sl(_rowshift(f, 1))
    nl = sl(_colshift(f, -1, is_w0, is_wlast))
    nr = sl(_colshift(f, 1, is_w0, is_wlast))

    def affw(n):
        d = fc - n
        return jnp.exp(jnp.sum(d * d, axis=0, keepdims=True) * neg_inv_mu)

    cpos = jax.lax.broadcasted_iota(jnp.int32, (1, STRIPF), 1)
    gpos = cpos + s * STRIPF
    cw = cpos % W
    wu = jnp.where(gpos < W, jnp.float32(0.0), affw(nu))
    wd = jnp.where(gpos >= HW - W, jnp.float32(0.0), affw(nd))
    wl = jnp.where(cw == 0, jnp.float32(0.0), affw(nl))
    wr = jnp.where(cw == W - 1, jnp.float32(0.0), affw(nr))
    deg = wu + wd + wl + wr
    aff_ref[0] = jnp.concatenate([wu, wd, wl, wr, deg], axis=0)


def _rsh(z, d):
    nb = z.shape[0]
    if d == 1:
        return jnp.concatenate([z[:, 1:, :], jnp.zeros((nb, 1, W), z.dtype)], axis=1)
    else:
        return jnp.concatenate([jnp.zeros((nb, 1, W), z.dtype), z[:, :-1, :]], axis=1)


def _csh(z, d):
    nb = z.shape[0]
    if d == 1:
        return jnp.concatenate([z[:, :, 1:], jnp.zeros((nb, H, 1), z.dtype)], axis=2)
    else:
        return jnp.concatenate([jnp.zeros((nb, H, 1), z.dtype), z[:, :, :-1]], axis=2)


def _cg_body(lam_ref, aff_ref, src_ref, mask_ref, y_ref, r_ref, p_ref):
    lam = lam_ref[0, 0]
    aff = aff_ref[...]
    wu = aff[:, 0]
    wd = aff[:, 1]
    wl = aff[:, 2]
    wr = aff[:, 3]
    deg = aff[:, 4]
    src = src_ref[...]
    msk = mask_ref[...]

    rr_i = jax.lax.broadcasted_iota(jnp.int32, (H, HL), 0)
    cc_i = jax.lax.broadcasted_iota(jnp.int32, (H, HL), 1)
    E = jnp.where(rr_i // S == cc_i, jnp.float32(1.0 / S), jnp.float32(0.0))
    rr2 = jax.lax.broadcasted_iota(jnp.int32, (HL, H), 0)
    cc2 = jax.lax.broadcasted_iota(jnp.int32, (HL, H), 1)
    ET = jnp.where(cc2 // S == rr2, jnp.float32(1.0 / S), jnp.float32(0.0))

    def up(m_b):
        return _DOT(E, _DOT(m_b, ET))

    def down(y_b):
        return _DOT(ET, _DOT(y_b, E))

    def down_up(y):
        return jnp.stack([lam * up(msk[b] * down(y[b])) for b in range(B)], axis=0)

    def Aop(y):
        Ly = deg * y - (wu * _rsh(y, -1) + wd * _rsh(y, 1)
                        + wl * _csh(y, -1) + wr * _csh(y, 1))
        return Ly + down_up(y)

    bvec = jnp.stack([lam * up(msk[b] * src[b]) for b in range(B)], axis=0)
    x0 = jnp.stack([float(S * S) * up(src[b]) for b in range(B)], axis=0)

    r0 = bvec - Aop(x0)
    y_ref[...] = x0
    r_ref[...] = r0
    p_ref[...] = r0
    rs0 = jnp.sum(r0 * r0)

    def body(i, rs):
        p = p_ref[...]
        Ap = Aop(p)
        alpha = rs / (jnp.sum(p * Ap) + 1e-12)
        y_ref[...] = y_ref[...] + alpha * p
        rnew = r_ref[...] - alpha * Ap
        r_ref[...] = rnew
        rs_new = jnp.sum(rnew * rnew)
        p_ref[...] = rnew + (rs_new / (rs + 1e-12)) * p
        return rs_new

    jax.lax.fori_loop(0, 30, body, rs0)


@jax.jit
def kernel(guide, source, mask_lr, y_bicubic, var_w, var_b, fe_w1, fe_b1,
           fe_w2, fe_b2, fe_w3, fe_b3, log_lambda, log_mu):
    f32 = jnp.float32
    x0 = jnp.concatenate([guide, y_bicubic], axis=1)
    x0 = jnp.pad(x0, ((0, 0), (0, 0), (HALO, HALO), (0, 0)))
    x0 = x0.reshape(B, 4, (H + 2 * HALO) * W)

    w1full = jnp.concatenate([var_w, fe_w1], axis=0)
    w1full = jnp.pad(w1full, ((0, 7), (0, 0), (0, 0), (0, 0)))
    w1r = jnp.transpose(w1full, (2, 0, 3, 1)).reshape(3, 40, 12)
    b1c = jnp.concatenate([var_b, fe_b1, jnp.zeros((7,), f32)]).reshape(40, 1)
    w2r = jnp.transpose(fe_w2, (2, 0, 3, 1)).reshape(3, 32, 96)
    w3r = jnp.transpose(fe_w3, (2, 0, 3, 1)).reshape(3, 32, 96)
    b2c = fe_b2.reshape(32, 1)
    b3c = fe_b3.reshape(32, 1)

    mu_arr = jnp.exp(log_mu).reshape(1, 1)
    lam_arr = jnp.exp(log_lambda).reshape(1, 1)

    var_flat, aff_flat = pl.pallas_call(
        _conv_aff_body,
        grid=(B, NSTRIP),
        in_specs=[
            pl.BlockSpec((1, 1), lambda b, s: (0, 0), memory_space=pltpu.SMEM),
            pl.BlockSpec((1, 4, (H + 2 * HALO) * W), lambda b, s: (b, 0, 0)),
            pl.BlockSpec((3, 40, 12), lambda b, s: (0, 0, 0)),
            pl.BlockSpec((40, 1), lambda b, s: (0, 0)),
            pl.BlockSpec((3, 32, 96), lambda b, s: (0, 0, 0)),
            pl.BlockSpec((32, 1), lambda b, s: (0, 0)),
            pl.BlockSpec((3, 32, 96), lambda b, s: (0, 0, 0)),
            pl.BlockSpec((32, 1), lambda b, s: (0, 0)),
        ],
        out_specs=[
            pl.BlockSpec((1, 1, STRIPF), lambda b, s: (b, 0, s)),
            pl.BlockSpec((1, 5, STRIPF), lambda b, s: (b, 0, s)),
        ],
        out_shape=[
            jax.ShapeDtypeStruct((B, 1, HW), f32),
            jax.ShapeDtypeStruct((B, 5, HW), f32),
        ],
    )(mu_arr, x0, w1r, b1c, w2r, b2c, w3r, b3c)

    var = var_flat.reshape(B, 1, H, W)
    aff = aff_flat.reshape(B, 5, H, W)

    y = pl.pallas_call(
        _cg_body,
        in_specs=[
            pl.BlockSpec(memory_space=pltpu.SMEM),
            pl.BlockSpec(memory_space=pltpu.VMEM),
            pl.BlockSpec(memory_space=pltpu.VMEM),
            pl.BlockSpec(memory_space=pltpu.VMEM),
        ],
        out_specs=pl.BlockSpec(memory_space=pltpu.VMEM),
        out_shape=jax.ShapeDtypeStruct((B, H, W), f32),
        scratch_shapes=[
            pltpu.VMEM((B, H, W), f32),
            pltpu.VMEM((B, H, W), f32),
        ],
    )(lam_arr, aff, source.reshape(B, HL, WL), mask_lr.reshape(B, HL, WL))

    return (y.reshape(B, 1, H, W), var, aff)
